# Initial kernel scaffold; baseline (speedup 1.0000x reference)
import jax
import jax.numpy as jnp
from jax.experimental import pallas as pl


def kernel(x, W):
    raise NotImplementedError("write your pallas kernel here")


# speedup vs baseline: 2.3917x; 2.3917x over previous
import functools

import jax
import jax.numpy as jnp
from jax.experimental import pallas as pl
from jax.experimental.pallas import tpu as pltpu


def _router_body(x_ref, wt_ref, wout_ref, iout_ref, loss_ref,
                 cacc, pacc, zacc, *, n_tokens, n_experts):
    step = pl.program_id(0)
    logits = jnp.dot(x_ref[:], wt_ref[:], preferred_element_type=jnp.float32)
    iota = jax.lax.broadcasted_iota(jnp.int32, logits.shape, 1)

    m1 = jnp.max(logits, axis=1, keepdims=True)
    i1 = jnp.min(jnp.where(logits == m1, iota, n_experts), axis=1,
                 keepdims=True)
    hit1 = iota == i1
    masked = jnp.where(hit1, -jnp.inf, logits)
    m2 = jnp.max(masked, axis=1, keepdims=True)
    i2 = jnp.min(jnp.where(masked == m2, iota, n_experts), axis=1,
                 keepdims=True)

    e2 = jnp.exp(m2 - m1)
    w1 = 1.0 / (1.0 + e2)
    w2 = e2 / (1.0 + e2)
    wout_ref[:] = jnp.concatenate([w1, w2], axis=1)
    iout_ref[:] = jnp.concatenate([i1, i2], axis=1)

    ex = jnp.exp(logits - m1)
    probs = ex / jnp.sum(ex, axis=1, keepdims=True)

    bc = jnp.sum(hit1.astype(jnp.float32), axis=0, keepdims=True)
    bp = jnp.sum(probs, axis=0, keepdims=True)
    bz = jnp.sum(logits * logits).reshape(1, 1)

    @pl.when(step == 0)
    def _init():
        cacc[:] = bc
        pacc[:] = bp
        zacc[:] = bz

    @pl.when(step != 0)
    def _accum():
        cacc[:] += bc
        pacc[:] += bp
        zacc[:] += bz

    @pl.when(step == pl.num_programs(0) - 1)
    def _finalize():
        aux = (n_experts / (n_tokens * n_tokens)) * jnp.sum(cacc[:] * pacc[:])
        z = jnp.sum(zacc[:]) * (0.001 / (n_tokens * n_experts))
        loss_ref[:] = jnp.full((1, 1), aux + z, dtype=jnp.float32)


def kernel(x, W):
    B, S, D = x.shape
    E = W.shape[0]
    N = B * S
    xf = x.reshape(N, D)
    wt = W.T

    T = min(2048, N)
    grid = (N // T,)

    body = functools.partial(_router_body, n_tokens=N, n_experts=E)
    wout, iout, loss = pl.pallas_call(
        body,
        grid=grid,
        in_specs=[
            pl.BlockSpec((T, D), lambda i: (i, 0)),
            pl.BlockSpec((D, E), lambda i: (0, 0)),
        ],
        out_specs=[
            pl.BlockSpec((T, 2), lambda i: (i, 0)),
            pl.BlockSpec((T, 2), lambda i: (i, 0)),
            pl.BlockSpec((1, 1), lambda i: (0, 0)),
        ],
        out_shape=[
            jax.ShapeDtypeStruct((N, 2), jnp.float32),
            jax.ShapeDtypeStruct((N, 2), jnp.int32),
            jax.ShapeDtypeStruct((1, 1), jnp.float32),
        ],
        scratch_shapes=[
            pltpu.VMEM((1, E), jnp.float32),
            pltpu.VMEM((1, E), jnp.float32),
            pltpu.VMEM((1, 1), jnp.float32),
        ],
        compiler_params=pltpu.CompilerParams(
            dimension_semantics=("arbitrary",),
        ),
    )(xf, wt)

    return (wout.reshape(B, S, 2), iout.reshape(B, S, 2), loss[0, 0])


# speedup vs baseline: 4.6611x; 1.9488x over previous
import functools

import jax
import jax.numpy as jnp
from jax.experimental import pallas as pl
from jax.experimental.pallas import tpu as pltpu


def _router_body(x_ref, wt_ref, wout_ref, iout_ref, loss_ref,
                 cacc, pacc, zacc, *, n_tokens, n_experts):
    step = pl.program_id(0)
    logits = jnp.dot(x_ref[:], wt_ref[:], preferred_element_type=jnp.float32)
    lt = logits.T
    iotaf = jax.lax.broadcasted_iota(jnp.int32, lt.shape, 0).astype(jnp.float32)

    m1 = jnp.max(lt, axis=0, keepdims=True)
    i1 = jnp.min(jnp.where(lt == m1, iotaf, float(n_experts)), axis=0,
                 keepdims=True)
    hit1 = iotaf == i1
    masked = jnp.where(hit1, -jnp.inf, lt)
    m2 = jnp.max(masked, axis=0, keepdims=True)
    i2 = jnp.min(jnp.where(masked == m2, iotaf, float(n_experts)), axis=0,
                 keepdims=True)

    e2 = jnp.exp(m2 - m1)
    rs = 1.0 / (1.0 + e2)
    w12 = jnp.concatenate([rs, e2 * rs], axis=0)
    i12 = jnp.concatenate([i1, i2], axis=0).astype(jnp.int32)
    wout_ref[:] = w12.reshape(wout_ref.shape)
    iout_ref[:] = i12.reshape(iout_ref.shape)

    ex = jnp.exp(lt - m1)
    scaled = ex * (1.0 / jnp.sum(ex, axis=0, keepdims=True))

    @pl.when(step == 0)
    def _init():
        cacc[:] = hit1.astype(jnp.float32)
        pacc[:] = scaled
        zacc[:] = lt * lt

    @pl.when(step != 0)
    def _accum():
        cacc[:] += hit1.astype(jnp.float32)
        pacc[:] += scaled
        zacc[:] += lt * lt

    @pl.when(step == pl.num_programs(0) - 1)
    def _finalize():
        c = jnp.sum(cacc[:], axis=1)
        p = jnp.sum(pacc[:], axis=1)
        aux = (n_experts / (n_tokens * n_tokens)) * jnp.sum(c * p)
        z = jnp.sum(zacc[:]) * (0.001 / (n_tokens * n_experts))
        loss_ref[:] = jnp.full((1, 1), aux + z, dtype=jnp.float32)


def kernel(x, W):
    B, S, D = x.shape
    E = W.shape[0]
    N = B * S
    xf = x.reshape(N, D)
    wt = W.T

    T = min(2048, N)
    nb = N // T
    grid = (nb,)

    body = functools.partial(_router_body, n_tokens=N, n_experts=E)
    wout, iout, loss = pl.pallas_call(
        body,
        grid=grid,
        in_specs=[
            pl.BlockSpec((T, D), lambda i: (i, 0)),
            pl.BlockSpec((D, E), lambda i: (0, 0)),
        ],
        out_specs=[
            pl.BlockSpec((1, 2, T), lambda i: (i, 0, 0)),
            pl.BlockSpec((1, 2, T), lambda i: (i, 0, 0)),
            pl.BlockSpec((1, 1), lambda i: (0, 0)),
        ],
        out_shape=[
            jax.ShapeDtypeStruct((nb, 2, T), jnp.float32),
            jax.ShapeDtypeStruct((nb, 2, T), jnp.int32),
            jax.ShapeDtypeStruct((1, 1), jnp.float32),
        ],
        scratch_shapes=[
            pltpu.VMEM((E, T), jnp.float32),
            pltpu.VMEM((E, T), jnp.float32),
            pltpu.VMEM((E, T), jnp.float32),
        ],
        compiler_params=pltpu.CompilerParams(
            dimension_semantics=("arbitrary",),
        ),
    )(xf, wt)

    wout = wout.transpose(0, 2, 1).reshape(B, S, 2)
    iout = iout.transpose(0, 2, 1).reshape(B, S, 2)
    return (wout, iout, loss[0, 0])


# speedup vs baseline: 5.0046x; 1.0737x over previous
import functools

import jax
import jax.numpy as jnp
from jax.experimental import pallas as pl
from jax.experimental.pallas import tpu as pltpu


def _router_body(x_ref, wt_ref, wout_ref, iout_ref, loss_ref,
                 cacc, pacc, zacc, *, n_tokens, n_experts):
    step = pl.program_id(0)
    logits = jnp.dot(x_ref[:], wt_ref[:], preferred_element_type=jnp.float32)
    lt = logits.T
    iotaf = jax.lax.broadcasted_iota(jnp.int32, lt.shape, 0).astype(jnp.float32)

    m1 = jnp.max(lt, axis=0, keepdims=True)
    i1 = jnp.min(jnp.where(lt == m1, iotaf, float(n_experts)), axis=0,
                 keepdims=True)
    hit1 = iotaf == i1
    masked = jnp.where(hit1, -jnp.inf, lt)
    m2 = jnp.max(masked, axis=0, keepdims=True)
    i2 = jnp.min(jnp.where(masked == m2, iotaf, float(n_experts)), axis=0,
                 keepdims=True)

    e2 = jnp.exp(m2 - m1)
    rs = 1.0 / (1.0 + e2)
    w12 = jnp.concatenate([rs, e2 * rs], axis=0)
    i12 = jnp.concatenate([i1, i2], axis=0).astype(jnp.int32)
    wout_ref[:] = w12.reshape(wout_ref.shape)
    iout_ref[:] = i12.reshape(iout_ref.shape)

    ex = jnp.exp(lt - m1)
    scaled = ex * (1.0 / jnp.sum(ex, axis=0, keepdims=True))

    @pl.when(step == 0)
    def _init():
        cacc[:] = hit1.astype(jnp.float32)
        pacc[:] = scaled
        zacc[:] = lt * lt

    @pl.when(step != 0)
    def _accum():
        cacc[:] += hit1.astype(jnp.float32)
        pacc[:] += scaled
        zacc[:] += lt * lt

    @pl.when(step == pl.num_programs(0) - 1)
    def _finalize():
        c = jnp.sum(cacc[:], axis=1)
        p = jnp.sum(pacc[:], axis=1)
        aux = (n_experts / (n_tokens * n_tokens)) * jnp.sum(c * p)
        z = jnp.sum(zacc[:]) * (0.001 / (n_tokens * n_experts))
        loss_ref[:] = jnp.full((1, 1), aux + z, dtype=jnp.float32)


def kernel(x, W):
    B, S, D = x.shape
    E = W.shape[0]
    N = B * S
    xf = x.reshape(N, D)
    wt = W.T

    T = min(4096, N)
    nb = N // T
    grid = (nb,)

    body = functools.partial(_router_body, n_tokens=N, n_experts=E)
    wout, iout, loss = pl.pallas_call(
        body,
        grid=grid,
        in_specs=[
            pl.BlockSpec((T, D), lambda i: (i, 0)),
            pl.BlockSpec((D, E), lambda i: (0, 0)),
        ],
        out_specs=[
            pl.BlockSpec((1, 2, T), lambda i: (i, 0, 0)),
            pl.BlockSpec((1, 2, T), lambda i: (i, 0, 0)),
            pl.BlockSpec((1, 1), lambda i: (0, 0)),
        ],
        out_shape=[
            jax.ShapeDtypeStruct((nb, 2, T), jnp.float32),
            jax.ShapeDtypeStruct((nb, 2, T), jnp.int32),
            jax.ShapeDtypeStruct((1, 1), jnp.float32),
        ],
        scratch_shapes=[
            pltpu.VMEM((E, T), jnp.float32),
            pltpu.VMEM((E, T), jnp.float32),
            pltpu.VMEM((E, T), jnp.float32),
        ],
        compiler_params=pltpu.CompilerParams(
            dimension_semantics=("arbitrary",),
        ),
    )(xf, wt)

    wout = wout.transpose(0, 2, 1).reshape(B, S, 2)
    iout = iout.transpose(0, 2, 1).reshape(B, S, 2)
    return (wout, iout, loss[0, 0])
